# baseline (speedup 1.0000x reference)
import functools

import numpy as np

import jax
import jax.numpy as jnp
from jax import lax
from jax.experimental import pallas as pl
from jax.experimental.pallas import tpu as pltpu
from jax.experimental.pallas import tpu_sc as plsc

N = 10000
D = 128
E = 160000
NCORE = 2
NSUB = 16
NW = NCORE * NSUB
EPT = E // NSUB
CHUNK = 32
NFULL = (EPT // CHUNK) & ~1
NPAIR = NFULL // 2
PARTIAL = EPT - NFULL * CHUNK
RPT = 624
TAIL = N - NSUB * RPT
SCALE = 0.25
LANES = 16

SIGMA_PACK = np.empty(128, np.int32)
for _dd in range(4):
    for _i in range(16):
        SIGMA_PACK[_dd * 32 + 2 * _i] = _dd * 32 + _i
        SIGMA_PACK[_dd * 32 + 2 * _i + 1] = _dd * 32 + 16 + _i

BLK = 1000
NBLK = (2 * N) // BLK
HALF_BLKS = N // BLK


def _proj_body(xkv_ref, xq_ref, wk_ref, bk_ref, wq_ref, bq_ref,
               wv_ref, bv_ref, kv_ref, q_ref):
    xkv = xkv_ref[...]
    xq = xq_ref[...]
    k = jnp.dot(xkv, wk_ref[0], preferred_element_type=jnp.float32) + bk_ref[0]
    v = jnp.dot(xkv, wv_ref[0], preferred_element_type=jnp.float32) + bv_ref[0]
    q = jnp.dot(xq, wq_ref[0], preferred_element_type=jnp.float32) + bq_ref[0]
    kv_ref[...] = jnp.concatenate([k, v], axis=1).astype(jnp.bfloat16)
    q_ref[...] = q


def _out_body(acc_ref, wo_ref, bo_ref, g_ref, b_ref, h_ref):
    h = jnp.dot(acc_ref[...], wo_ref[...],
                preferred_element_type=jnp.float32) + bo_ref[...]
    mu = jnp.mean(h, axis=-1, keepdims=True)
    dlt = h - mu
    var = jnp.mean(dlt * dlt, axis=-1, keepdims=True)
    h_ref[...] = dlt * lax.rsqrt(var + 1e-5) * g_ref[...] + b_ref[...]


def _sc_body(kv_hbm, q_hbm, ui_hbm, iu_hbm, out_hbm,
             sq_a, sq_b, dst_a, dst_b, kv_a, kv_b, q_a, q_b, m_a, m_b,
             sq_p, dst_p, abuf, acc_sh,
             semi_a, semi_b, semk_a, semk_b, semq_a, semq_b,
             semd_a, semd_b, sems_a, sems_b):
    c = lax.axis_index("c")
    s = lax.axis_index("s")
    lanes = lax.iota(jnp.int32, LANES)
    coff = c * N

    def _zrow(i, carry):
        for dd in range(D // LANES):
            m_a[i, pl.ds(dd * LANES, LANES)] = jnp.zeros((LANES,), jnp.float32)
        return carry

    lax.fori_loop(0, CHUNK, _zrow, 0)
    r0 = s * RPT
    for z in range(RPT // CHUNK):
        pltpu.sync_copy(m_a, acc_sh.at[pl.ds(r0 + z * CHUNK, CHUNK)])
    zrem = RPT - (RPT // CHUNK) * CHUNK
    if zrem:
        pltpu.sync_copy(m_a.at[pl.ds(0, zrem)],
                        acc_sh.at[pl.ds(r0 + (RPT // CHUNK) * CHUNK, zrem)])

    @pl.when(s == 0)
    def _zero_tail():
        pltpu.sync_copy(m_a.at[pl.ds(0, TAIL)],
                        acc_sh.at[pl.ds(NSUB * RPT, TAIL)])

    plsc.subcore_barrier()

    def _issue_idx(t, sqx, semi, count):
        off = s * EPT + t * CHUNK

        @pl.when(c == 0)
        def _():
            pltpu.async_copy(ui_hbm.at[pl.ds(off, count)],
                             sqx.at[0, pl.ds(0, count)], semi)
            pltpu.async_copy(ui_hbm.at[pl.ds(E + off, count)],
                             sqx.at[1, pl.ds(0, count)], semi)

        @pl.when(c == 1)
        def _():
            pltpu.async_copy(iu_hbm.at[pl.ds(off, count)],
                             sqx.at[0, pl.ds(0, count)], semi)
            pltpu.async_copy(iu_hbm.at[pl.ds(E + off, count)],
                             sqx.at[1, pl.ds(0, count)], semi)

    def _wait_adjust_idx(sqx, semi, count):
        for r2 in range(2):
            pltpu.make_async_copy(ui_hbm.at[pl.ds(0, count)],
                                  sqx.at[r2, pl.ds(0, count)], semi).wait()
        for r2 in range(2):
            for k2 in range(count // LANES):
                sl = pl.ds(k2 * LANES, LANES)
                sqx[r2, sl] = sqx[r2, sl] + coff

    def _issue_dst(t, dstx, semd, count):
        off = s * EPT + t * CHUNK

        @pl.when(c == 0)
        def _():
            pltpu.async_copy(ui_hbm.at[pl.ds(E + off, count)], dstx, semd)

        @pl.when(c == 1)
        def _():
            pltpu.async_copy(iu_hbm.at[pl.ds(E + off, count)], dstx, semd)

    def _issue_g(sqx, kvx, qx, semk, semq, count):
        pltpu.async_copy(kv_hbm.at[sqx.at[0, pl.ds(0, count)]],
                         kvx.at[pl.ds(0, count)], semk)
        pltpu.async_copy(q_hbm.at[sqx.at[1, pl.ds(0, count)]],
                         qx.at[pl.ds(0, count)], semq)

    def _wait_g(sqx, kvx, qx, semk, semq, count):
        pltpu.make_async_copy(kv_hbm.at[sqx.at[0, pl.ds(0, count)]],
                              kvx.at[pl.ds(0, count)], semk).wait()
        pltpu.make_async_copy(q_hbm.at[sqx.at[1, pl.ds(0, count)]],
                              qx.at[pl.ds(0, count)], semq).wait()

    def _group(kvx, qx, mx, base):
        for e in range(LANES):
            row = base + e
            acc = None
            for dd in range(D // (2 * LANES)):
                kb = plsc.bitcast(kvx[row, pl.ds(dd * LANES, LANES)],
                                  jnp.bfloat16)
                k0, k1 = plsc.unpack(kb, format=plsc.PackFormat.INTERLEAVED)
                pp = (k0 * qx[row, pl.ds(dd * 2 * LANES, LANES)]
                      + k1 * qx[row, pl.ds(dd * 2 * LANES + LANES, LANES)])
                acc = pp if acc is None else acc + pp
            abuf[e, :] = acc
        tot = plsc.load_gather(
            abuf, [lanes, jnp.zeros((LANES,), jnp.int32)])
        for l in range(1, LANES):
            tot = tot + plsc.load_gather(
                abuf, [lanes, jnp.full((LANES,), l, jnp.int32)])
        asig = 1.0 / (1.0 + jnp.exp(tot * (-SCALE)))
        for e in range(LANES):
            av = asig[e]
            row = base + e
            for dd in range(D // (2 * LANES)):
                vb = plsc.bitcast(
                    kvx[row, pl.ds(D // 2 + dd * LANES, LANES)], jnp.bfloat16)
                v0, v1 = plsc.unpack(vb, format=plsc.PackFormat.INTERLEAVED)
                mx[row, pl.ds(dd * 2 * LANES, LANES)] = av * v0
                mx[row, pl.ds(dd * 2 * LANES + LANES, LANES)] = av * v1

    def _half(j, t, sq_cur, sq_oth, kv_cur, q_cur, m_cur, dst_cur,
              semi_cur, semi_oth, semk_cur, semq_cur, semk_oth, semq_oth,
              semd_cur, sems_cur, kv_oth, q_oth):
        @pl.when(t + 1 < NFULL)
        def _start_next():
            _wait_adjust_idx(sq_oth, semi_oth, CHUNK)
            _issue_g(sq_oth, kv_oth, q_oth, semk_oth, semq_oth, CHUNK)

        _wait_g(sq_cur, kv_cur, q_cur, semk_cur, semq_cur, CHUNK)

        @pl.when(t + 2 < NFULL)
        def _prefetch_idx():
            _issue_idx(t + 2, sq_cur, semi_cur, CHUNK)

        @pl.when(j > 0)
        def _drain_scatter():
            pltpu.make_async_copy(m_cur, acc_sh.at[dst_cur], sems_cur).wait()

        _issue_dst(t, dst_cur, semd_cur, CHUNK)
        _group(kv_cur, q_cur, m_cur, 0)
        _group(kv_cur, q_cur, m_cur, LANES)
        pltpu.make_async_copy(ui_hbm.at[pl.ds(0, CHUNK)], dst_cur,
                              semd_cur).wait()
        pltpu.async_copy(m_cur, acc_sh.at[dst_cur], sems_cur, add=True)

    _issue_idx(0, sq_a, semi_a, CHUNK)
    _issue_idx(1, sq_b, semi_b, CHUNK)
    _wait_adjust_idx(sq_a, semi_a, CHUNK)
    _issue_g(sq_a, kv_a, q_a, semk_a, semq_a, CHUNK)

    def _pair(j, carry):
        t = 2 * j
        _half(j, t, sq_a, sq_b, kv_a, q_a, m_a, dst_a, semi_a, semi_b,
              semk_a, semq_a, semk_b, semq_b, semd_a, sems_a, kv_b, q_b)
        _half(j, t + 1, sq_b, sq_a, kv_b, q_b, m_b, dst_b, semi_b, semi_a,
              semk_b, semq_b, semk_a, semq_a, semd_b, sems_b, kv_a, q_a)
        return carry

    lax.fori_loop(0, NPAIR, _pair, 0)
    pltpu.make_async_copy(m_a, acc_sh.at[dst_a], sems_a).wait()
    pltpu.make_async_copy(m_b, acc_sh.at[dst_b], sems_b).wait()

    if PARTIAL:
        _issue_idx(NFULL, sq_p, semi_a, PARTIAL)
        _issue_dst(NFULL, dst_p, semd_a, PARTIAL)
        _wait_adjust_idx(sq_p, semi_a, PARTIAL)
        _issue_g(sq_p, kv_a, q_a, semk_a, semq_a, PARTIAL)
        _wait_g(sq_p, kv_a, q_a, semk_a, semq_a, PARTIAL)
        pltpu.make_async_copy(ui_hbm.at[pl.ds(0, PARTIAL)], dst_p,
                              semd_a).wait()
        _group(kv_a, q_a, m_a, 0)
        pltpu.sync_copy(m_a.at[pl.ds(0, PARTIAL)], acc_sh.at[dst_p], add=True)

    plsc.subcore_barrier()

    pltpu.sync_copy(acc_sh.at[pl.ds(r0, RPT)],
                    out_hbm.at[pl.ds(c * N + r0, RPT)])

    @pl.when(s == 0)
    def _write_tail():
        pltpu.sync_copy(acc_sh.at[pl.ds(NSUB * RPT, TAIL)],
                        out_hbm.at[pl.ds(c * N + NSUB * RPT, TAIL)])


_sc_call = functools.partial(
    pl.kernel,
    out_type=jax.ShapeDtypeStruct((2 * N, D), jnp.float32),
    mesh=plsc.VectorSubcoreMesh(core_axis_name="c", subcore_axis_name="s",
                                num_cores=NCORE, num_subcores=NSUB),
    compiler_params=pltpu.CompilerParams(needs_layout_passes=False),
    scratch_types=[
        pltpu.VMEM((2, CHUNK), jnp.int32),
        pltpu.VMEM((2, CHUNK), jnp.int32),
        pltpu.VMEM((CHUNK,), jnp.int32),
        pltpu.VMEM((CHUNK,), jnp.int32),
        pltpu.VMEM((CHUNK, D), jnp.float32),
        pltpu.VMEM((CHUNK, D), jnp.float32),
        pltpu.VMEM((CHUNK, D), jnp.float32),
        pltpu.VMEM((CHUNK, D), jnp.float32),
        pltpu.VMEM((CHUNK, D), jnp.float32),
        pltpu.VMEM((CHUNK, D), jnp.float32),
        pltpu.VMEM((2, LANES), jnp.int32),
        pltpu.VMEM((LANES,), jnp.int32),
        pltpu.VMEM((LANES, LANES), jnp.float32),
        pltpu.VMEM_SHARED((N, D), jnp.float32),
        pltpu.SemaphoreType.DMA,
        pltpu.SemaphoreType.DMA,
        pltpu.SemaphoreType.DMA,
        pltpu.SemaphoreType.DMA,
        pltpu.SemaphoreType.DMA,
        pltpu.SemaphoreType.DMA,
        pltpu.SemaphoreType.DMA,
        pltpu.SemaphoreType.DMA,
        pltpu.SemaphoreType.DMA,
        pltpu.SemaphoreType.DMA,
    ],
)(_sc_body)


def kernel(x_user, x_item, edge_index_ui, edge_index_iu,
           W_K_ui, b_K_ui, W_Q_ui, b_Q_ui, W_V_ui, b_V_ui,
           W_K_iu, b_K_iu, W_Q_iu, b_Q_iu, W_V_iu, b_V_iu,
           W_O, b_O, ln_gamma, ln_beta):
    f32 = jnp.float32
    xkv = jnp.concatenate([x_user, x_item], axis=0).astype(f32)
    xq = jnp.concatenate([x_item, x_user], axis=0).astype(f32)
    sp = jnp.asarray(SIGMA_PACK)
    wk = jnp.stack([W_K_ui, W_K_iu])[:, :, sp]
    wq = jnp.stack([W_Q_ui, W_Q_iu])
    wv = jnp.stack([W_V_ui, W_V_iu])[:, :, sp]
    bk = jnp.stack([b_K_ui, b_K_iu])[:, sp].reshape(2, 1, D)
    bq = jnp.stack([b_Q_ui, b_Q_iu]).reshape(2, 1, D)
    bv = jnp.stack([b_V_ui, b_V_iu])[:, sp].reshape(2, 1, D)

    wspec = pl.BlockSpec((1, D, D), lambda i: (i // HALF_BLKS, 0, 0))
    bspec = pl.BlockSpec((1, 1, D), lambda i: (i // HALF_BLKS, 0, 0))
    rspec = pl.BlockSpec((BLK, D), lambda i: (i, 0))
    kv, q = pl.pallas_call(
        _proj_body,
        grid=(NBLK,),
        in_specs=[rspec, rspec, wspec, bspec, wspec, bspec, wspec, bspec],
        out_specs=[pl.BlockSpec((BLK, 2 * D), lambda i: (i, 0)), rspec],
        out_shape=[jax.ShapeDtypeStruct((2 * N, 2 * D), jnp.bfloat16),
                   jax.ShapeDtypeStruct((2 * N, D), f32)],
    )(xkv, xq, wk, bk, wq, bq, wv, bv)

    i32 = jnp.int32
    kv_words = jax.lax.bitcast_convert_type(
        kv.reshape(2 * N, D, 2), jnp.float32)
    acc = _sc_call(kv_words, q,
                   edge_index_ui.astype(i32).reshape(-1),
                   edge_index_iu.astype(i32).reshape(-1))

    vspec = pl.BlockSpec((1, D), lambda i: (0, 0))
    h = pl.pallas_call(
        _out_body,
        grid=(NBLK,),
        in_specs=[rspec, pl.BlockSpec((D, D), lambda i: (0, 0)),
                  vspec, vspec, vspec],
        out_specs=rspec,
        out_shape=jax.ShapeDtypeStruct((2 * N, D), f32),
    )(acc, W_O.astype(f32), b_O.reshape(1, D), ln_gamma.reshape(1, D),
      ln_beta.reshape(1, D))

    return h[N:], h[:N]


# speedup vs baseline: 1.4099x; 1.4099x over previous
import functools

import numpy as np

import jax
import jax.numpy as jnp
from jax import lax
from jax.experimental import pallas as pl
from jax.experimental.pallas import tpu as pltpu
from jax.experimental.pallas import tpu_sc as plsc

N = 10000
D = 128
E = 160000
NCORE = 2
NSUB = 16
NW = NCORE * NSUB
EPT = E // NSUB
CHUNK = 32
NFULL = (EPT // CHUNK) & ~1
NPAIR = NFULL // 2
PARTIAL = EPT - NFULL * CHUNK
RPT = 624
TAIL = N - NSUB * RPT
SCALE = 0.25
LANES = 16


BLK = 1000
NBLK = (2 * N) // BLK
HALF_BLKS = N // BLK


def _pack_words(a):
    ab = a.astype(jnp.bfloat16)
    lo = jax.lax.bitcast_convert_type(ab[:, :D // 2],
                                      jnp.uint16).astype(jnp.uint32)
    hi = jax.lax.bitcast_convert_type(ab[:, D // 2:],
                                      jnp.uint16).astype(jnp.uint32)
    return jax.lax.bitcast_convert_type(lo | (hi << 16), jnp.float32)


def _proj_body(xkv_ref, xq_ref, wk_ref, bk_ref, wq_ref, bq_ref,
               wv_ref, bv_ref, kv_ref, q_ref):
    xkv = xkv_ref[...]
    xq = xq_ref[...]
    k = jnp.dot(xkv, wk_ref[0], preferred_element_type=jnp.float32) + bk_ref[0]
    v = jnp.dot(xkv, wv_ref[0], preferred_element_type=jnp.float32) + bv_ref[0]
    q = jnp.dot(xq, wq_ref[0], preferred_element_type=jnp.float32) + bq_ref[0]
    kv_ref[...] = jnp.concatenate([_pack_words(k), _pack_words(v)], axis=1)
    q_ref[...] = q


def _out_body(acc_ref, wo_ref, bo_ref, g_ref, b_ref, h_ref):
    h = jnp.dot(acc_ref[...], wo_ref[...],
                preferred_element_type=jnp.float32) + bo_ref[...]
    mu = jnp.mean(h, axis=-1, keepdims=True)
    dlt = h - mu
    var = jnp.mean(dlt * dlt, axis=-1, keepdims=True)
    h_ref[...] = dlt * lax.rsqrt(var + 1e-5) * g_ref[...] + b_ref[...]


def _sc_body(kv_hbm, q_hbm, ui_hbm, iu_hbm, out_hbm,
             sq_a, sq_b, dst_a, dst_b, kv_a, kv_b, q_a, q_b, m_a, m_b,
             sq_p, dst_p, abuf, acc_sh,
             semi_a, semi_b, semk_a, semk_b, semq_a, semq_b,
             semd_a, semd_b, sems_a, sems_b):
    c = lax.axis_index("c")
    s = lax.axis_index("s")
    lanes = lax.iota(jnp.int32, LANES)
    coff = c * N

    def _zrow(i, carry):
        for dd in range(D // LANES):
            m_a[i, pl.ds(dd * LANES, LANES)] = jnp.zeros((LANES,), jnp.float32)
        return carry

    lax.fori_loop(0, CHUNK, _zrow, 0)
    r0 = s * RPT
    for z in range(RPT // CHUNK):
        pltpu.sync_copy(m_a, acc_sh.at[pl.ds(r0 + z * CHUNK, CHUNK)])
    zrem = RPT - (RPT // CHUNK) * CHUNK
    if zrem:
        pltpu.sync_copy(m_a.at[pl.ds(0, zrem)],
                        acc_sh.at[pl.ds(r0 + (RPT // CHUNK) * CHUNK, zrem)])

    @pl.when(s == 0)
    def _zero_tail():
        pltpu.sync_copy(m_a.at[pl.ds(0, TAIL)],
                        acc_sh.at[pl.ds(NSUB * RPT, TAIL)])

    plsc.subcore_barrier()

    def _issue_idx(t, sqx, semi, count):
        off = s * EPT + t * CHUNK

        @pl.when(c == 0)
        def _():
            pltpu.async_copy(ui_hbm.at[pl.ds(off, count)],
                             sqx.at[0, pl.ds(0, count)], semi)
            pltpu.async_copy(ui_hbm.at[pl.ds(E + off, count)],
                             sqx.at[1, pl.ds(0, count)], semi)

        @pl.when(c == 1)
        def _():
            pltpu.async_copy(iu_hbm.at[pl.ds(off, count)],
                             sqx.at[0, pl.ds(0, count)], semi)
            pltpu.async_copy(iu_hbm.at[pl.ds(E + off, count)],
                             sqx.at[1, pl.ds(0, count)], semi)

    def _wait_adjust_idx(sqx, semi, count):
        for r2 in range(2):
            pltpu.make_async_copy(ui_hbm.at[pl.ds(0, count)],
                                  sqx.at[r2, pl.ds(0, count)], semi).wait()
        for r2 in range(2):
            for k2 in range(count // LANES):
                sl = pl.ds(k2 * LANES, LANES)
                sqx[r2, sl] = sqx[r2, sl] + coff

    def _issue_dst(t, dstx, semd, count):
        off = s * EPT + t * CHUNK

        @pl.when(c == 0)
        def _():
            pltpu.async_copy(ui_hbm.at[pl.ds(E + off, count)], dstx, semd)

        @pl.when(c == 1)
        def _():
            pltpu.async_copy(iu_hbm.at[pl.ds(E + off, count)], dstx, semd)

    def _issue_g(sqx, kvx, qx, semk, semq, count):
        pltpu.async_copy(kv_hbm.at[sqx.at[0, pl.ds(0, count)]],
                         kvx.at[pl.ds(0, count)], semk)
        pltpu.async_copy(q_hbm.at[sqx.at[1, pl.ds(0, count)]],
                         qx.at[pl.ds(0, count)], semq)

    def _wait_g(sqx, kvx, qx, semk, semq, count):
        pltpu.make_async_copy(kv_hbm.at[sqx.at[0, pl.ds(0, count)]],
                              kvx.at[pl.ds(0, count)], semk).wait()
        pltpu.make_async_copy(q_hbm.at[sqx.at[1, pl.ds(0, count)]],
                              qx.at[pl.ds(0, count)], semq).wait()

    def _group(kvx, qx, mx, base):
        for e in range(LANES):
            row = base + e
            acc = None
            for dd in range(D // (2 * LANES)):
                kb = plsc.bitcast(kvx[row, pl.ds(dd * LANES, LANES)],
                                  jnp.bfloat16)
                k0, k1 = plsc.unpack(kb, format=plsc.PackFormat.INTERLEAVED)
                pp = (k0 * qx[row, pl.ds(dd * LANES, LANES)]
                      + k1 * qx[row, pl.ds(D // 2 + dd * LANES, LANES)])
                acc = pp if acc is None else acc + pp
            abuf[e, :] = acc
        tot = plsc.load_gather(
            abuf, [lanes, jnp.zeros((LANES,), jnp.int32)])
        for l in range(1, LANES):
            tot = tot + plsc.load_gather(
                abuf, [lanes, jnp.full((LANES,), l, jnp.int32)])
        asig = 1.0 / (1.0 + jnp.exp(tot * (-SCALE)))
        for e in range(LANES):
            av = asig[e]
            row = base + e
            for dd in range(D // (2 * LANES)):
                vb = plsc.bitcast(
                    kvx[row, pl.ds(D // 2 + dd * LANES, LANES)], jnp.bfloat16)
                v0, v1 = plsc.unpack(vb, format=plsc.PackFormat.INTERLEAVED)
                mx[row, pl.ds(dd * LANES, LANES)] = av * v0
                mx[row, pl.ds(D // 2 + dd * LANES, LANES)] = av * v1

    def _half(j, t, sq_cur, sq_oth, kv_cur, q_cur, m_cur, dst_cur,
              semi_cur, semi_oth, semk_cur, semq_cur, semk_oth, semq_oth,
              semd_cur, sems_cur, kv_oth, q_oth):
        @pl.when(t + 1 < NFULL)
        def _start_next():
            _wait_adjust_idx(sq_oth, semi_oth, CHUNK)
            _issue_g(sq_oth, kv_oth, q_oth, semk_oth, semq_oth, CHUNK)

        _wait_g(sq_cur, kv_cur, q_cur, semk_cur, semq_cur, CHUNK)

        @pl.when(t + 2 < NFULL)
        def _prefetch_idx():
            _issue_idx(t + 2, sq_cur, semi_cur, CHUNK)

        @pl.when(j > 0)
        def _drain_scatter():
            pltpu.make_async_copy(m_cur, acc_sh.at[dst_cur], sems_cur).wait()

        _issue_dst(t, dst_cur, semd_cur, CHUNK)
        _group(kv_cur, q_cur, m_cur, 0)
        _group(kv_cur, q_cur, m_cur, LANES)
        pltpu.make_async_copy(ui_hbm.at[pl.ds(0, CHUNK)], dst_cur,
                              semd_cur).wait()
        pltpu.async_copy(m_cur, acc_sh.at[dst_cur], sems_cur, add=True)

    _issue_idx(0, sq_a, semi_a, CHUNK)
    _issue_idx(1, sq_b, semi_b, CHUNK)
    _wait_adjust_idx(sq_a, semi_a, CHUNK)
    _issue_g(sq_a, kv_a, q_a, semk_a, semq_a, CHUNK)

    def _pair(j, carry):
        t = 2 * j
        _half(j, t, sq_a, sq_b, kv_a, q_a, m_a, dst_a, semi_a, semi_b,
              semk_a, semq_a, semk_b, semq_b, semd_a, sems_a, kv_b, q_b)
        _half(j, t + 1, sq_b, sq_a, kv_b, q_b, m_b, dst_b, semi_b, semi_a,
              semk_b, semq_b, semk_a, semq_a, semd_b, sems_b, kv_a, q_a)
        return carry

    lax.fori_loop(0, NPAIR, _pair, 0)
    pltpu.make_async_copy(m_a, acc_sh.at[dst_a], sems_a).wait()
    pltpu.make_async_copy(m_b, acc_sh.at[dst_b], sems_b).wait()

    if PARTIAL:
        _issue_idx(NFULL, sq_p, semi_a, PARTIAL)
        _issue_dst(NFULL, dst_p, semd_a, PARTIAL)
        _wait_adjust_idx(sq_p, semi_a, PARTIAL)
        _issue_g(sq_p, kv_a, q_a, semk_a, semq_a, PARTIAL)
        _wait_g(sq_p, kv_a, q_a, semk_a, semq_a, PARTIAL)
        pltpu.make_async_copy(ui_hbm.at[pl.ds(0, PARTIAL)], dst_p,
                              semd_a).wait()
        _group(kv_a, q_a, m_a, 0)
        pltpu.sync_copy(m_a.at[pl.ds(0, PARTIAL)], acc_sh.at[dst_p], add=True)

    plsc.subcore_barrier()

    pltpu.sync_copy(acc_sh.at[pl.ds(r0, RPT)],
                    out_hbm.at[pl.ds(c * N + r0, RPT)])

    @pl.when(s == 0)
    def _write_tail():
        pltpu.sync_copy(acc_sh.at[pl.ds(NSUB * RPT, TAIL)],
                        out_hbm.at[pl.ds(c * N + NSUB * RPT, TAIL)])


_sc_call = functools.partial(
    pl.kernel,
    out_type=jax.ShapeDtypeStruct((2 * N, D), jnp.float32),
    mesh=plsc.VectorSubcoreMesh(core_axis_name="c", subcore_axis_name="s",
                                num_cores=NCORE, num_subcores=NSUB),
    compiler_params=pltpu.CompilerParams(needs_layout_passes=False),
    scratch_types=[
        pltpu.VMEM((2, CHUNK), jnp.int32),
        pltpu.VMEM((2, CHUNK), jnp.int32),
        pltpu.VMEM((CHUNK,), jnp.int32),
        pltpu.VMEM((CHUNK,), jnp.int32),
        pltpu.VMEM((CHUNK, D), jnp.float32),
        pltpu.VMEM((CHUNK, D), jnp.float32),
        pltpu.VMEM((CHUNK, D), jnp.float32),
        pltpu.VMEM((CHUNK, D), jnp.float32),
        pltpu.VMEM((CHUNK, D), jnp.float32),
        pltpu.VMEM((CHUNK, D), jnp.float32),
        pltpu.VMEM((2, LANES), jnp.int32),
        pltpu.VMEM((LANES,), jnp.int32),
        pltpu.VMEM((LANES, LANES), jnp.float32),
        pltpu.VMEM_SHARED((N, D), jnp.float32),
        pltpu.SemaphoreType.DMA,
        pltpu.SemaphoreType.DMA,
        pltpu.SemaphoreType.DMA,
        pltpu.SemaphoreType.DMA,
        pltpu.SemaphoreType.DMA,
        pltpu.SemaphoreType.DMA,
        pltpu.SemaphoreType.DMA,
        pltpu.SemaphoreType.DMA,
        pltpu.SemaphoreType.DMA,
        pltpu.SemaphoreType.DMA,
    ],
)(_sc_body)


def kernel(x_user, x_item, edge_index_ui, edge_index_iu,
           W_K_ui, b_K_ui, W_Q_ui, b_Q_ui, W_V_ui, b_V_ui,
           W_K_iu, b_K_iu, W_Q_iu, b_Q_iu, W_V_iu, b_V_iu,
           W_O, b_O, ln_gamma, ln_beta):
    f32 = jnp.float32
    xkv = jnp.concatenate([x_user, x_item], axis=0).astype(f32)
    xq = jnp.concatenate([x_item, x_user], axis=0).astype(f32)
    wk = jnp.stack([W_K_ui, W_K_iu])
    wq = jnp.stack([W_Q_ui, W_Q_iu])
    wv = jnp.stack([W_V_ui, W_V_iu])
    bk = jnp.stack([b_K_ui, b_K_iu]).reshape(2, 1, D)
    bq = jnp.stack([b_Q_ui, b_Q_iu]).reshape(2, 1, D)
    bv = jnp.stack([b_V_ui, b_V_iu]).reshape(2, 1, D)

    wspec = pl.BlockSpec((1, D, D), lambda i: (i // HALF_BLKS, 0, 0))
    bspec = pl.BlockSpec((1, 1, D), lambda i: (i // HALF_BLKS, 0, 0))
    rspec = pl.BlockSpec((BLK, D), lambda i: (i, 0))
    kv, q = pl.pallas_call(
        _proj_body,
        grid=(NBLK,),
        in_specs=[rspec, rspec, wspec, bspec, wspec, bspec, wspec, bspec],
        out_specs=[rspec, rspec],
        out_shape=[jax.ShapeDtypeStruct((2 * N, D), f32),
                   jax.ShapeDtypeStruct((2 * N, D), f32)],
    )(xkv, xq, wk, bk, wq, bq, wv, bv)

    i32 = jnp.int32
    acc = _sc_call(kv, q,
                   edge_index_ui.astype(i32).reshape(-1),
                   edge_index_iu.astype(i32).reshape(-1))

    vspec = pl.BlockSpec((1, D), lambda i: (0, 0))
    h = pl.pallas_call(
        _out_body,
        grid=(NBLK,),
        in_specs=[rspec, pl.BlockSpec((D, D), lambda i: (0, 0)),
                  vspec, vspec, vspec],
        out_specs=rspec,
        out_shape=jax.ShapeDtypeStruct((2 * N, D), f32),
    )(acc, W_O.astype(f32), b_O.reshape(1, D), ln_gamma.reshape(1, D),
      ln_beta.reshape(1, D))

    return h[N:], h[:N]


# speedup vs baseline: 1.4471x; 1.0263x over previous
import functools

import numpy as np

import jax
import jax.numpy as jnp
from jax import lax
from jax.experimental import pallas as pl
from jax.experimental.pallas import tpu as pltpu
from jax.experimental.pallas import tpu_sc as plsc

N = 10000
D = 128
E = 160000
NCORE = 2
NSUB = 16
NW = NCORE * NSUB
EPT = E // NSUB
CHUNK = 32
NFULL = (EPT // CHUNK) & ~1
NPAIR = NFULL // 2
PARTIAL = EPT - NFULL * CHUNK
RPT = 624
TAIL = N - NSUB * RPT
SCALE = 0.25
LANES = 16


BLK = 1000
NBLK = (2 * N) // BLK
HALF_BLKS = N // BLK


def _pack_words(a):
    ab = a.astype(jnp.bfloat16)
    lo = jax.lax.bitcast_convert_type(ab[:, :D // 2],
                                      jnp.uint16).astype(jnp.uint32)
    hi = jax.lax.bitcast_convert_type(ab[:, D // 2:],
                                      jnp.uint16).astype(jnp.uint32)
    return jax.lax.bitcast_convert_type(lo | (hi << 16), jnp.float32)


def _proj_body(xu_ref, xi_ref, wk_ref, bk_ref, wq_ref, bq_ref,
               wv_ref, bv_ref, kv_ref, q_ref):
    first = pl.program_id(0) < HALF_BLKS
    xkv = jnp.where(first, xu_ref[...], xi_ref[...])
    xq = jnp.where(first, xi_ref[...], xu_ref[...])
    k = jnp.dot(xkv, wk_ref[0], preferred_element_type=jnp.float32) + bk_ref[0]
    v = jnp.dot(xkv, wv_ref[0], preferred_element_type=jnp.float32) + bv_ref[0]
    q = jnp.dot(xq, wq_ref[0], preferred_element_type=jnp.float32) + bq_ref[0]
    kv_ref[...] = jnp.concatenate([_pack_words(k), _pack_words(v)], axis=1)
    q_ref[...] = q


def _out_body(acc_ref, wo_ref, bo_ref, g_ref, b_ref, h_ref):
    h = jnp.dot(acc_ref[...], wo_ref[...],
                preferred_element_type=jnp.float32) + bo_ref[...]
    mu = jnp.mean(h, axis=-1, keepdims=True)
    dlt = h - mu
    var = jnp.mean(dlt * dlt, axis=-1, keepdims=True)
    h_ref[...] = dlt * lax.rsqrt(var + 1e-5) * g_ref[...] + b_ref[...]


def _sc_body(kv_hbm, q_hbm, ui_hbm, iu_hbm, out_hbm,
             sq_a, sq_b, dst_a, dst_b, kv_a, kv_b, q_a, q_b, m_a, m_b,
             sq_p, dst_p, abuf, acc_sh,
             semi_a, semi_b, semk_a, semk_b, semq_a, semq_b,
             semd_a, semd_b, sems_a, sems_b):
    c = lax.axis_index("c")
    s = lax.axis_index("s")
    lanes = lax.iota(jnp.int32, LANES)
    coff = c * N

    def _zrow(i, carry):
        for dd in range(D // LANES):
            m_a[i, pl.ds(dd * LANES, LANES)] = jnp.zeros((LANES,), jnp.float32)
        return carry

    lax.fori_loop(0, CHUNK, _zrow, 0)
    r0 = s * RPT
    for z in range(RPT // CHUNK):
        pltpu.sync_copy(m_a, acc_sh.at[pl.ds(r0 + z * CHUNK, CHUNK)])
    zrem = RPT - (RPT // CHUNK) * CHUNK
    if zrem:
        pltpu.sync_copy(m_a.at[pl.ds(0, zrem)],
                        acc_sh.at[pl.ds(r0 + (RPT // CHUNK) * CHUNK, zrem)])

    @pl.when(s == 0)
    def _zero_tail():
        pltpu.sync_copy(m_a.at[pl.ds(0, TAIL)],
                        acc_sh.at[pl.ds(NSUB * RPT, TAIL)])

    plsc.subcore_barrier()

    def _issue_idx(t, sqx, semi, count):
        off = s * EPT + t * CHUNK

        @pl.when(c == 0)
        def _():
            pltpu.async_copy(ui_hbm.at[pl.ds(off, count)],
                             sqx.at[0, pl.ds(0, count)], semi)
            pltpu.async_copy(ui_hbm.at[pl.ds(E + off, count)],
                             sqx.at[1, pl.ds(0, count)], semi)

        @pl.when(c == 1)
        def _():
            pltpu.async_copy(iu_hbm.at[pl.ds(off, count)],
                             sqx.at[0, pl.ds(0, count)], semi)
            pltpu.async_copy(iu_hbm.at[pl.ds(E + off, count)],
                             sqx.at[1, pl.ds(0, count)], semi)

    def _wait_adjust_idx(sqx, semi, count):
        for r2 in range(2):
            pltpu.make_async_copy(ui_hbm.at[pl.ds(0, count)],
                                  sqx.at[r2, pl.ds(0, count)], semi).wait()
        for r2 in range(2):
            for k2 in range(count // LANES):
                sl = pl.ds(k2 * LANES, LANES)
                sqx[r2, sl] = sqx[r2, sl] + coff

    def _issue_dst(t, dstx, semd, count):
        off = s * EPT + t * CHUNK

        @pl.when(c == 0)
        def _():
            pltpu.async_copy(ui_hbm.at[pl.ds(E + off, count)], dstx, semd)

        @pl.when(c == 1)
        def _():
            pltpu.async_copy(iu_hbm.at[pl.ds(E + off, count)], dstx, semd)

    def _issue_g(sqx, kvx, qx, semk, semq, count):
        pltpu.async_copy(kv_hbm.at[sqx.at[0, pl.ds(0, count)]],
                         kvx.at[pl.ds(0, count)], semk)
        pltpu.async_copy(q_hbm.at[sqx.at[1, pl.ds(0, count)]],
                         qx.at[pl.ds(0, count)], semq)

    def _wait_g(sqx, kvx, qx, semk, semq, count):
        pltpu.make_async_copy(kv_hbm.at[sqx.at[0, pl.ds(0, count)]],
                              kvx.at[pl.ds(0, count)], semk).wait()
        pltpu.make_async_copy(q_hbm.at[sqx.at[1, pl.ds(0, count)]],
                              qx.at[pl.ds(0, count)], semq).wait()

    def _group(kvx, qx, mx, base):
        for e in range(LANES):
            row = base + e
            acc = None
            for dd in range(D // (2 * LANES)):
                kb = plsc.bitcast(kvx[row, pl.ds(dd * LANES, LANES)],
                                  jnp.bfloat16)
                k0, k1 = plsc.unpack(kb, format=plsc.PackFormat.INTERLEAVED)
                pp = (k0 * qx[row, pl.ds(dd * LANES, LANES)]
                      + k1 * qx[row, pl.ds(D // 2 + dd * LANES, LANES)])
                acc = pp if acc is None else acc + pp
            abuf[e, :] = acc
        tot = plsc.load_gather(
            abuf, [lanes, jnp.zeros((LANES,), jnp.int32)])
        for l in range(1, LANES):
            tot = tot + plsc.load_gather(
                abuf, [lanes, jnp.full((LANES,), l, jnp.int32)])
        asig = 1.0 / (1.0 + jnp.exp(tot * (-SCALE)))
        for e in range(LANES):
            av = asig[e]
            row = base + e
            for dd in range(D // (2 * LANES)):
                vb = plsc.bitcast(
                    kvx[row, pl.ds(D // 2 + dd * LANES, LANES)], jnp.bfloat16)
                v0, v1 = plsc.unpack(vb, format=plsc.PackFormat.INTERLEAVED)
                mx[row, pl.ds(dd * LANES, LANES)] = av * v0
                mx[row, pl.ds(D // 2 + dd * LANES, LANES)] = av * v1

    def _half(j, t, sq_cur, sq_oth, kv_cur, q_cur, m_cur, dst_cur,
              semi_cur, semi_oth, semk_cur, semq_cur, semk_oth, semq_oth,
              semd_cur, sems_cur, kv_oth, q_oth):
        @pl.when(t + 1 < NFULL)
        def _start_next():
            _wait_adjust_idx(sq_oth, semi_oth, CHUNK)
            _issue_g(sq_oth, kv_oth, q_oth, semk_oth, semq_oth, CHUNK)

        _wait_g(sq_cur, kv_cur, q_cur, semk_cur, semq_cur, CHUNK)

        @pl.when(t + 2 < NFULL)
        def _prefetch_idx():
            _issue_idx(t + 2, sq_cur, semi_cur, CHUNK)

        @pl.when(j > 0)
        def _drain_scatter():
            pltpu.make_async_copy(m_cur, acc_sh.at[dst_cur], sems_cur).wait()

        _issue_dst(t, dst_cur, semd_cur, CHUNK)
        _group(kv_cur, q_cur, m_cur, 0)
        _group(kv_cur, q_cur, m_cur, LANES)
        pltpu.make_async_copy(ui_hbm.at[pl.ds(0, CHUNK)], dst_cur,
                              semd_cur).wait()
        pltpu.async_copy(m_cur, acc_sh.at[dst_cur], sems_cur, add=True)

    _issue_idx(0, sq_a, semi_a, CHUNK)
    _issue_idx(1, sq_b, semi_b, CHUNK)
    _wait_adjust_idx(sq_a, semi_a, CHUNK)
    _issue_g(sq_a, kv_a, q_a, semk_a, semq_a, CHUNK)

    def _pair(j, carry):
        t = 2 * j
        _half(j, t, sq_a, sq_b, kv_a, q_a, m_a, dst_a, semi_a, semi_b,
              semk_a, semq_a, semk_b, semq_b, semd_a, sems_a, kv_b, q_b)
        _half(j, t + 1, sq_b, sq_a, kv_b, q_b, m_b, dst_b, semi_b, semi_a,
              semk_b, semq_b, semk_a, semq_a, semd_b, sems_b, kv_a, q_a)
        return carry

    lax.fori_loop(0, NPAIR, _pair, 0)
    pltpu.make_async_copy(m_a, acc_sh.at[dst_a], sems_a).wait()
    pltpu.make_async_copy(m_b, acc_sh.at[dst_b], sems_b).wait()

    if PARTIAL:
        _issue_idx(NFULL, sq_p, semi_a, PARTIAL)
        _issue_dst(NFULL, dst_p, semd_a, PARTIAL)
        _wait_adjust_idx(sq_p, semi_a, PARTIAL)
        _issue_g(sq_p, kv_a, q_a, semk_a, semq_a, PARTIAL)
        _wait_g(sq_p, kv_a, q_a, semk_a, semq_a, PARTIAL)
        pltpu.make_async_copy(ui_hbm.at[pl.ds(0, PARTIAL)], dst_p,
                              semd_a).wait()
        _group(kv_a, q_a, m_a, 0)
        pltpu.sync_copy(m_a.at[pl.ds(0, PARTIAL)], acc_sh.at[dst_p], add=True)

    plsc.subcore_barrier()

    pltpu.sync_copy(acc_sh.at[pl.ds(r0, RPT)],
                    out_hbm.at[pl.ds(c * N + r0, RPT)])

    @pl.when(s == 0)
    def _write_tail():
        pltpu.sync_copy(acc_sh.at[pl.ds(NSUB * RPT, TAIL)],
                        out_hbm.at[pl.ds(c * N + NSUB * RPT, TAIL)])


_sc_call = functools.partial(
    pl.kernel,
    out_type=jax.ShapeDtypeStruct((2 * N, D), jnp.float32),
    mesh=plsc.VectorSubcoreMesh(core_axis_name="c", subcore_axis_name="s",
                                num_cores=NCORE, num_subcores=NSUB),
    compiler_params=pltpu.CompilerParams(needs_layout_passes=False),
    scratch_types=[
        pltpu.VMEM((2, CHUNK), jnp.int32),
        pltpu.VMEM((2, CHUNK), jnp.int32),
        pltpu.VMEM((CHUNK,), jnp.int32),
        pltpu.VMEM((CHUNK,), jnp.int32),
        pltpu.VMEM((CHUNK, D), jnp.float32),
        pltpu.VMEM((CHUNK, D), jnp.float32),
        pltpu.VMEM((CHUNK, D), jnp.float32),
        pltpu.VMEM((CHUNK, D), jnp.float32),
        pltpu.VMEM((CHUNK, D), jnp.float32),
        pltpu.VMEM((CHUNK, D), jnp.float32),
        pltpu.VMEM((2, LANES), jnp.int32),
        pltpu.VMEM((LANES,), jnp.int32),
        pltpu.VMEM((LANES, LANES), jnp.float32),
        pltpu.VMEM_SHARED((N, D), jnp.float32),
        pltpu.SemaphoreType.DMA,
        pltpu.SemaphoreType.DMA,
        pltpu.SemaphoreType.DMA,
        pltpu.SemaphoreType.DMA,
        pltpu.SemaphoreType.DMA,
        pltpu.SemaphoreType.DMA,
        pltpu.SemaphoreType.DMA,
        pltpu.SemaphoreType.DMA,
        pltpu.SemaphoreType.DMA,
        pltpu.SemaphoreType.DMA,
    ],
)(_sc_body)


def kernel(x_user, x_item, edge_index_ui, edge_index_iu,
           W_K_ui, b_K_ui, W_Q_ui, b_Q_ui, W_V_ui, b_V_ui,
           W_K_iu, b_K_iu, W_Q_iu, b_Q_iu, W_V_iu, b_V_iu,
           W_O, b_O, ln_gamma, ln_beta):
    f32 = jnp.float32
    wk = jnp.stack([W_K_ui, W_K_iu])
    wq = jnp.stack([W_Q_ui, W_Q_iu])
    wv = jnp.stack([W_V_ui, W_V_iu])
    bk = jnp.stack([b_K_ui, b_K_iu]).reshape(2, 1, D)
    bq = jnp.stack([b_Q_ui, b_Q_iu]).reshape(2, 1, D)
    bv = jnp.stack([b_V_ui, b_V_iu]).reshape(2, 1, D)

    wspec = pl.BlockSpec((1, D, D), lambda i: (i // HALF_BLKS, 0, 0))
    bspec = pl.BlockSpec((1, 1, D), lambda i: (i // HALF_BLKS, 0, 0))
    rspec = pl.BlockSpec((BLK, D), lambda i: (i, 0))
    hspec = pl.BlockSpec((BLK, D), lambda i: (i % HALF_BLKS, 0))
    kv, q = pl.pallas_call(
        _proj_body,
        grid=(NBLK,),
        in_specs=[hspec, hspec, wspec, bspec, wspec, bspec, wspec, bspec],
        out_specs=[rspec, rspec],
        out_shape=[jax.ShapeDtypeStruct((2 * N, D), f32),
                   jax.ShapeDtypeStruct((2 * N, D), f32)],
    )(x_user.astype(f32), x_item.astype(f32), wk, bk, wq, bq, wv, bv)

    i32 = jnp.int32
    acc = _sc_call(kv, q,
                   edge_index_ui.astype(i32).reshape(-1),
                   edge_index_iu.astype(i32).reshape(-1))

    vspec = pl.BlockSpec((1, D), lambda i: (0, 0))
    h = pl.pallas_call(
        _out_body,
        grid=(NBLK,),
        in_specs=[rspec, pl.BlockSpec((D, D), lambda i: (0, 0)),
                  vspec, vspec, vspec],
        out_specs=rspec,
        out_shape=jax.ShapeDtypeStruct((2 * N, D), f32),
    )(acc, W_O.astype(f32), b_O.reshape(1, D), ln_gamma.reshape(1, D),
      ln_beta.reshape(1, D))

    return h[N:], h[:N]


# speedup vs baseline: 1.5335x; 1.0597x over previous
import functools

import numpy as np

import jax
import jax.numpy as jnp
from jax import lax
from jax.experimental import pallas as pl
from jax.experimental.pallas import tpu as pltpu
from jax.experimental.pallas import tpu_sc as plsc

N = 10000
D = 128
E = 160000
NCORE = 2
NSUB = 16
NW = NCORE * NSUB
EPT = E // NSUB
CHUNK = 32
NFULL = (EPT // CHUNK) & ~1
NPAIR = NFULL // 2
PARTIAL = EPT - NFULL * CHUNK
RPT = 624
TAIL = N - NSUB * RPT
SCALE = 0.25
LANES = 16


BLK = 1000
NBLK = (2 * N) // BLK
HALF_BLKS = N // BLK


def _pack_words(a):
    ab = a.astype(jnp.bfloat16)
    lo = jax.lax.bitcast_convert_type(ab[:, :D // 2],
                                      jnp.uint16).astype(jnp.uint32)
    hi = jax.lax.bitcast_convert_type(ab[:, D // 2:],
                                      jnp.uint16).astype(jnp.uint32)
    return jax.lax.bitcast_convert_type(lo | (hi << 16), jnp.float32)


def _proj_body(xu_ref, xi_ref, wk_ref, bk_ref, wq_ref, bq_ref,
               wv_ref, bv_ref, kv_ref, q_ref):
    first = pl.program_id(0) < HALF_BLKS
    xkv = jnp.where(first, xu_ref[...], xi_ref[...])
    xq = jnp.where(first, xi_ref[...], xu_ref[...])
    k = jnp.dot(xkv, wk_ref[0], preferred_element_type=jnp.float32) + bk_ref[0]
    v = jnp.dot(xkv, wv_ref[0], preferred_element_type=jnp.float32) + bv_ref[0]
    q = jnp.dot(xq, wq_ref[0], preferred_element_type=jnp.float32) + bq_ref[0]
    kv_ref[...] = jnp.concatenate([_pack_words(k), _pack_words(v)], axis=1)
    q_ref[...] = q


def _out_body(acc_ref, wo_ref, bo_ref, g_ref, b_ref, h_ref):
    h = jnp.dot(acc_ref[...], wo_ref[...],
                preferred_element_type=jnp.float32) + bo_ref[...]
    mu = jnp.mean(h, axis=-1, keepdims=True)
    dlt = h - mu
    var = jnp.mean(dlt * dlt, axis=-1, keepdims=True)
    h_ref[...] = dlt * lax.rsqrt(var + 1e-5) * g_ref[...] + b_ref[...]


def _sc_body(kv_hbm, q_hbm, ui_hbm, iu_hbm, out_hbm,
             sq_a, sq_b, sq_c, sq_d, dst_a, dst_b,
             kv_a, kv_b, kv_c, kv_d, q_a, q_b, q_c, q_d, m_a, m_b,
             sq_p, dst_p, abuf, acc_sh,
             semi_a, semi_b, semi_c, semi_d,
             semk_a, semk_b, semk_c, semk_d,
             semq_a, semq_b, semq_c, semq_d,
             semd_a, semd_b, sems_a, sems_b):
    c = lax.axis_index("c")
    s = lax.axis_index("s")
    lanes = lax.iota(jnp.int32, LANES)
    coff = c * N

    def _zrow(i, carry):
        for dd in range(D // LANES):
            m_a[i, pl.ds(dd * LANES, LANES)] = jnp.zeros((LANES,), jnp.float32)
        return carry

    lax.fori_loop(0, CHUNK, _zrow, 0)
    r0 = s * RPT
    for z in range(RPT // CHUNK):
        pltpu.sync_copy(m_a, acc_sh.at[pl.ds(r0 + z * CHUNK, CHUNK)])
    zrem = RPT - (RPT // CHUNK) * CHUNK
    if zrem:
        pltpu.sync_copy(m_a.at[pl.ds(0, zrem)],
                        acc_sh.at[pl.ds(r0 + (RPT // CHUNK) * CHUNK, zrem)])

    @pl.when(s == 0)
    def _zero_tail():
        pltpu.sync_copy(m_a.at[pl.ds(0, TAIL)],
                        acc_sh.at[pl.ds(NSUB * RPT, TAIL)])

    plsc.subcore_barrier()

    def _issue_idx(t, sqx, semi, count):
        off = s * EPT + t * CHUNK

        @pl.when(c == 0)
        def _():
            pltpu.async_copy(ui_hbm.at[pl.ds(off, count)],
                             sqx.at[0, pl.ds(0, count)], semi)
            pltpu.async_copy(ui_hbm.at[pl.ds(E + off, count)],
                             sqx.at[1, pl.ds(0, count)], semi)

        @pl.when(c == 1)
        def _():
            pltpu.async_copy(iu_hbm.at[pl.ds(off, count)],
                             sqx.at[0, pl.ds(0, count)], semi)
            pltpu.async_copy(iu_hbm.at[pl.ds(E + off, count)],
                             sqx.at[1, pl.ds(0, count)], semi)

    def _wait_adjust_idx(sqx, semi, count):
        for r2 in range(2):
            pltpu.make_async_copy(ui_hbm.at[pl.ds(0, count)],
                                  sqx.at[r2, pl.ds(0, count)], semi).wait()
        for r2 in range(2):
            for k2 in range(count // LANES):
                sl = pl.ds(k2 * LANES, LANES)
                sqx[r2, sl] = sqx[r2, sl] + coff

    def _issue_dst(t, dstx, semd, count):
        off = s * EPT + t * CHUNK

        @pl.when(c == 0)
        def _():
            pltpu.async_copy(ui_hbm.at[pl.ds(E + off, count)], dstx, semd)

        @pl.when(c == 1)
        def _():
            pltpu.async_copy(iu_hbm.at[pl.ds(E + off, count)], dstx, semd)

    def _issue_g(sqx, kvx, qx, semk, semq, count):
        pltpu.async_copy(kv_hbm.at[sqx.at[0, pl.ds(0, count)]],
                         kvx.at[pl.ds(0, count)], semk)
        pltpu.async_copy(q_hbm.at[sqx.at[1, pl.ds(0, count)]],
                         qx.at[pl.ds(0, count)], semq)

    def _wait_g(sqx, kvx, qx, semk, semq, count):
        pltpu.make_async_copy(kv_hbm.at[sqx.at[0, pl.ds(0, count)]],
                              kvx.at[pl.ds(0, count)], semk).wait()
        pltpu.make_async_copy(q_hbm.at[sqx.at[1, pl.ds(0, count)]],
                              qx.at[pl.ds(0, count)], semq).wait()

    def _group(kvx, qx, mx, base):
        for e in range(LANES):
            row = base + e
            acc = None
            for dd in range(D // (2 * LANES)):
                kb = plsc.bitcast(kvx[row, pl.ds(dd * LANES, LANES)],
                                  jnp.bfloat16)
                k0, k1 = plsc.unpack(kb, format=plsc.PackFormat.INTERLEAVED)
                pp = (k0 * qx[row, pl.ds(dd * LANES, LANES)]
                      + k1 * qx[row, pl.ds(D // 2 + dd * LANES, LANES)])
                acc = pp if acc is None else acc + pp
            abuf[e, :] = acc
        tot = plsc.load_gather(
            abuf, [lanes, jnp.zeros((LANES,), jnp.int32)])
        for l in range(1, LANES):
            tot = tot + plsc.load_gather(
                abuf, [lanes, jnp.full((LANES,), l, jnp.int32)])
        asig = 1.0 / (1.0 + jnp.exp(tot * (-SCALE)))
        for e in range(LANES):
            av = asig[e]
            row = base + e
            for dd in range(D // (2 * LANES)):
                vb = plsc.bitcast(
                    kvx[row, pl.ds(D // 2 + dd * LANES, LANES)], jnp.bfloat16)
                v0, v1 = plsc.unpack(vb, format=plsc.PackFormat.INTERLEAVED)
                mx[row, pl.ds(dd * LANES, LANES)] = av * v0
                mx[row, pl.ds(D // 2 + dd * LANES, LANES)] = av * v1

    def _half(t, sqc, kvc, qc, semic, semkc, semqc,
              sqn, kvn, qn, semin, semkn, semqn,
              m_cur, dst_cur, semd_cur, sems_cur):
        @pl.when(t + 2 < NFULL)
        def _start_next():
            _wait_adjust_idx(sqn, semin, CHUNK)
            _issue_g(sqn, kvn, qn, semkn, semqn, CHUNK)

        _wait_g(sqc, kvc, qc, semkc, semqc, CHUNK)

        @pl.when(t + 4 < NFULL)
        def _prefetch_idx():
            _issue_idx(t + 4, sqc, semic, CHUNK)

        @pl.when(t >= 2)
        def _drain_scatter():
            pltpu.make_async_copy(m_cur, acc_sh.at[dst_cur], sems_cur).wait()

        _issue_dst(t, dst_cur, semd_cur, CHUNK)
        _group(kvc, qc, m_cur, 0)
        _group(kvc, qc, m_cur, LANES)
        pltpu.make_async_copy(ui_hbm.at[pl.ds(0, CHUNK)], dst_cur,
                              semd_cur).wait()
        pltpu.async_copy(m_cur, acc_sh.at[dst_cur], sems_cur, add=True)

    _issue_idx(0, sq_a, semi_a, CHUNK)
    _issue_idx(1, sq_b, semi_b, CHUNK)
    _issue_idx(2, sq_c, semi_c, CHUNK)
    _issue_idx(3, sq_d, semi_d, CHUNK)
    _wait_adjust_idx(sq_a, semi_a, CHUNK)
    _issue_g(sq_a, kv_a, q_a, semk_a, semq_a, CHUNK)
    _wait_adjust_idx(sq_b, semi_b, CHUNK)
    _issue_g(sq_b, kv_b, q_b, semk_b, semq_b, CHUNK)

    sets = ((sq_a, kv_a, q_a, semi_a, semk_a, semq_a),
            (sq_b, kv_b, q_b, semi_b, semk_b, semq_b),
            (sq_c, kv_c, q_c, semi_c, semk_c, semq_c),
            (sq_d, kv_d, q_d, semi_d, semk_d, semq_d))
    mset = ((m_a, dst_a, semd_a, sems_a), (m_b, dst_b, semd_b, sems_b))

    def _quad(j, carry):
        t0 = 4 * j
        for b in range(4):
            _half(t0 + b, *sets[b], *sets[(b + 2) % 4], *mset[b % 2])
        return carry

    lax.fori_loop(0, NFULL // 4, _quad, 0)
    pltpu.make_async_copy(m_a, acc_sh.at[dst_a], sems_a).wait()
    pltpu.make_async_copy(m_b, acc_sh.at[dst_b], sems_b).wait()

    if PARTIAL:
        _issue_idx(NFULL, sq_p, semi_a, PARTIAL)
        _issue_dst(NFULL, dst_p, semd_a, PARTIAL)
        _wait_adjust_idx(sq_p, semi_a, PARTIAL)
        _issue_g(sq_p, kv_a, q_a, semk_a, semq_a, PARTIAL)
        _wait_g(sq_p, kv_a, q_a, semk_a, semq_a, PARTIAL)
        pltpu.make_async_copy(ui_hbm.at[pl.ds(0, PARTIAL)], dst_p,
                              semd_a).wait()
        _group(kv_a, q_a, m_a, 0)
        pltpu.sync_copy(m_a.at[pl.ds(0, PARTIAL)], acc_sh.at[dst_p], add=True)

    plsc.subcore_barrier()

    pltpu.sync_copy(acc_sh.at[pl.ds(r0, RPT)],
                    out_hbm.at[pl.ds(c * N + r0, RPT)])

    @pl.when(s == 0)
    def _write_tail():
        pltpu.sync_copy(acc_sh.at[pl.ds(NSUB * RPT, TAIL)],
                        out_hbm.at[pl.ds(c * N + NSUB * RPT, TAIL)])


_sc_call = functools.partial(
    pl.kernel,
    out_type=jax.ShapeDtypeStruct((2 * N, D), jnp.float32),
    mesh=plsc.VectorSubcoreMesh(core_axis_name="c", subcore_axis_name="s",
                                num_cores=NCORE, num_subcores=NSUB),
    compiler_params=pltpu.CompilerParams(needs_layout_passes=False),
    scratch_types=[
        pltpu.VMEM((2, CHUNK), jnp.int32),
        pltpu.VMEM((2, CHUNK), jnp.int32),
        pltpu.VMEM((2, CHUNK), jnp.int32),
        pltpu.VMEM((2, CHUNK), jnp.int32),
        pltpu.VMEM((CHUNK,), jnp.int32),
        pltpu.VMEM((CHUNK,), jnp.int32),
        pltpu.VMEM((CHUNK, D), jnp.float32),
        pltpu.VMEM((CHUNK, D), jnp.float32),
        pltpu.VMEM((CHUNK, D), jnp.float32),
        pltpu.VMEM((CHUNK, D), jnp.float32),
        pltpu.VMEM((CHUNK, D), jnp.float32),
        pltpu.VMEM((CHUNK, D), jnp.float32),
        pltpu.VMEM((CHUNK, D), jnp.float32),
        pltpu.VMEM((CHUNK, D), jnp.float32),
        pltpu.VMEM((CHUNK, D), jnp.float32),
        pltpu.VMEM((CHUNK, D), jnp.float32),
        pltpu.VMEM((2, LANES), jnp.int32),
        pltpu.VMEM((LANES,), jnp.int32),
        pltpu.VMEM((LANES, LANES), jnp.float32),
        pltpu.VMEM_SHARED((N, D), jnp.float32),
        pltpu.SemaphoreType.DMA,
        pltpu.SemaphoreType.DMA,
        pltpu.SemaphoreType.DMA,
        pltpu.SemaphoreType.DMA,
        pltpu.SemaphoreType.DMA,
        pltpu.SemaphoreType.DMA,
        pltpu.SemaphoreType.DMA,
        pltpu.SemaphoreType.DMA,
        pltpu.SemaphoreType.DMA,
        pltpu.SemaphoreType.DMA,
        pltpu.SemaphoreType.DMA,
        pltpu.SemaphoreType.DMA,
        pltpu.SemaphoreType.DMA,
        pltpu.SemaphoreType.DMA,
        pltpu.SemaphoreType.DMA,
        pltpu.SemaphoreType.DMA,
    ],
)(_sc_body)


def kernel(x_user, x_item, edge_index_ui, edge_index_iu,
           W_K_ui, b_K_ui, W_Q_ui, b_Q_ui, W_V_ui, b_V_ui,
           W_K_iu, b_K_iu, W_Q_iu, b_Q_iu, W_V_iu, b_V_iu,
           W_O, b_O, ln_gamma, ln_beta):
    f32 = jnp.float32
    wk = jnp.stack([W_K_ui, W_K_iu])
    wq = jnp.stack([W_Q_ui, W_Q_iu])
    wv = jnp.stack([W_V_ui, W_V_iu])
    bk = jnp.stack([b_K_ui, b_K_iu]).reshape(2, 1, D)
    bq = jnp.stack([b_Q_ui, b_Q_iu]).reshape(2, 1, D)
    bv = jnp.stack([b_V_ui, b_V_iu]).reshape(2, 1, D)

    wspec = pl.BlockSpec((1, D, D), lambda i: (i // HALF_BLKS, 0, 0))
    bspec = pl.BlockSpec((1, 1, D), lambda i: (i // HALF_BLKS, 0, 0))
    rspec = pl.BlockSpec((BLK, D), lambda i: (i, 0))
    hspec = pl.BlockSpec((BLK, D), lambda i: (i % HALF_BLKS, 0))
    kv, q = pl.pallas_call(
        _proj_body,
        grid=(NBLK,),
        in_specs=[hspec, hspec, wspec, bspec, wspec, bspec, wspec, bspec],
        out_specs=[rspec, rspec],
        out_shape=[jax.ShapeDtypeStruct((2 * N, D), f32),
                   jax.ShapeDtypeStruct((2 * N, D), f32)],
    )(x_user.astype(f32), x_item.astype(f32), wk, bk, wq, bq, wv, bv)

    i32 = jnp.int32
    acc = _sc_call(kv, q,
                   edge_index_ui.astype(i32).reshape(-1),
                   edge_index_iu.astype(i32).reshape(-1))

    vspec = pl.BlockSpec((1, D), lambda i: (0, 0))
    h = pl.pallas_call(
        _out_body,
        grid=(NBLK,),
        in_specs=[rspec, pl.BlockSpec((D, D), lambda i: (0, 0)),
                  vspec, vspec, vspec],
        out_specs=rspec,
        out_shape=jax.ShapeDtypeStruct((2 * N, D), f32),
    )(acc, W_O.astype(f32), b_O.reshape(1, D), ln_gamma.reshape(1, D),
      ln_beta.reshape(1, D))

    return h[N:], h[:N]


# speedup vs baseline: 1.5672x; 1.0220x over previous
import functools

import numpy as np

import jax
import jax.numpy as jnp
from jax import lax
from jax.experimental import pallas as pl
from jax.experimental.pallas import tpu as pltpu
from jax.experimental.pallas import tpu_sc as plsc

N = 10000
D = 128
E = 160000
NCORE = 2
NSUB = 16
NW = NCORE * NSUB
EPT = E // NSUB
CHUNK = 32
NFULL = (EPT // CHUNK) & ~1
NPAIR = NFULL // 2
PARTIAL = EPT - NFULL * CHUNK
RPT = 624
TAIL = N - NSUB * RPT
SCALE = 0.25
LANES = 16


BLK = 1000
NBLK = (2 * N) // BLK
HALF_BLKS = N // BLK


def _pack_words(a):
    ab = a.astype(jnp.bfloat16)
    lo = jax.lax.bitcast_convert_type(ab[:, :D // 2],
                                      jnp.uint16).astype(jnp.uint32)
    hi = jax.lax.bitcast_convert_type(ab[:, D // 2:],
                                      jnp.uint16).astype(jnp.uint32)
    return jax.lax.bitcast_convert_type(lo | (hi << 16), jnp.float32)


def _proj_body(xu_ref, xi_ref, wk_ref, bk_ref, wq_ref, bq_ref,
               wv_ref, bv_ref, kv_ref, q_ref):
    first = pl.program_id(0) < HALF_BLKS
    xkv = jnp.where(first, xu_ref[...], xi_ref[...])
    xq = jnp.where(first, xi_ref[...], xu_ref[...])
    k = jnp.dot(xkv, wk_ref[0], preferred_element_type=jnp.float32) + bk_ref[0]
    v = jnp.dot(xkv, wv_ref[0], preferred_element_type=jnp.float32) + bv_ref[0]
    q = jnp.dot(xq, wq_ref[0], preferred_element_type=jnp.float32) + bq_ref[0]
    kv_ref[...] = jnp.concatenate([_pack_words(k), _pack_words(v)], axis=1)
    q_ref[...] = q


def _out_body(acc_ref, wo_ref, bo_ref, g_ref, b_ref, h_ref):
    h = jnp.dot(acc_ref[...], wo_ref[...],
                preferred_element_type=jnp.float32) + bo_ref[...]
    mu = jnp.mean(h, axis=-1, keepdims=True)
    dlt = h - mu
    var = jnp.mean(dlt * dlt, axis=-1, keepdims=True)
    h_ref[...] = dlt * lax.rsqrt(var + 1e-5) * g_ref[...] + b_ref[...]


def _sc_body(kv_hbm, q_hbm, ui_hbm, iu_hbm, out_hbm,
             sq_a, sq_b, sq_c, sq_d, dst_a, dst_b,
             kv_a, kv_b, kv_c, kv_d, q_a, q_b, q_c, q_d, m_a, m_b,
             sq_p, dst_p, abuf, acc_sh,
             semi_a, semi_b, semi_c, semi_d,
             semk_a, semk_b, semk_c, semk_d,
             semq_a, semq_b, semq_c, semq_d,
             semd_a, semd_b, sems_a, sems_b):
    c = lax.axis_index("c")
    s = lax.axis_index("s")
    lanes = lax.iota(jnp.int32, LANES)
    coff = c * N

    def _zrow(i, carry):
        for dd in range(D // LANES):
            m_a[i, pl.ds(dd * LANES, LANES)] = jnp.zeros((LANES,), jnp.float32)
        return carry

    lax.fori_loop(0, CHUNK, _zrow, 0)
    r0 = s * RPT
    for z in range(RPT // CHUNK):
        pltpu.sync_copy(m_a, acc_sh.at[pl.ds(r0 + z * CHUNK, CHUNK)])
    zrem = RPT - (RPT // CHUNK) * CHUNK
    if zrem:
        pltpu.sync_copy(m_a.at[pl.ds(0, zrem)],
                        acc_sh.at[pl.ds(r0 + (RPT // CHUNK) * CHUNK, zrem)])

    @pl.when(s == 0)
    def _zero_tail():
        pltpu.sync_copy(m_a.at[pl.ds(0, TAIL)],
                        acc_sh.at[pl.ds(NSUB * RPT, TAIL)])

    plsc.subcore_barrier()

    def _issue_idx(t, sqx, semi, count):
        off = s * EPT + t * CHUNK

        @pl.when(c == 0)
        def _():
            pltpu.async_copy(ui_hbm.at[pl.ds(off, count)],
                             sqx.at[0, pl.ds(0, count)], semi)
            pltpu.async_copy(ui_hbm.at[pl.ds(E + off, count)],
                             sqx.at[1, pl.ds(0, count)], semi)

        @pl.when(c == 1)
        def _():
            pltpu.async_copy(iu_hbm.at[pl.ds(off, count)],
                             sqx.at[0, pl.ds(0, count)], semi)
            pltpu.async_copy(iu_hbm.at[pl.ds(E + off, count)],
                             sqx.at[1, pl.ds(0, count)], semi)

    def _wait_adjust_idx(sqx, semi, count):
        for r2 in range(2):
            pltpu.make_async_copy(ui_hbm.at[pl.ds(0, count)],
                                  sqx.at[r2, pl.ds(0, count)], semi).wait()
        for r2 in range(2):
            for k2 in range(count // LANES):
                sl = pl.ds(k2 * LANES, LANES)
                sqx[r2, sl] = sqx[r2, sl] + coff

    def _issue_dst(t, dstx, semd, count):
        off = s * EPT + t * CHUNK

        @pl.when(c == 0)
        def _():
            pltpu.async_copy(ui_hbm.at[pl.ds(E + off, count)], dstx, semd)

        @pl.when(c == 1)
        def _():
            pltpu.async_copy(iu_hbm.at[pl.ds(E + off, count)], dstx, semd)

    def _issue_g(sqx, kvx, qx, semk, semq, count):
        pltpu.async_copy(kv_hbm.at[sqx.at[0, pl.ds(0, count)]],
                         kvx.at[pl.ds(0, count)], semk)
        pltpu.async_copy(q_hbm.at[sqx.at[1, pl.ds(0, count)]],
                         qx.at[pl.ds(0, count)], semq)

    def _wait_g(sqx, kvx, qx, semk, semq, count):
        pltpu.make_async_copy(kv_hbm.at[sqx.at[0, pl.ds(0, count)]],
                              kvx.at[pl.ds(0, count)], semk).wait()
        pltpu.make_async_copy(q_hbm.at[sqx.at[1, pl.ds(0, count)]],
                              qx.at[pl.ds(0, count)], semq).wait()

    def _group(kvx, qx, mx, base):
        for e in range(LANES):
            row = base + e
            acc = None
            for dd in range(D // (2 * LANES)):
                kb = plsc.bitcast(kvx[row, pl.ds(dd * LANES, LANES)],
                                  jnp.bfloat16)
                k0, k1 = plsc.unpack(kb, format=plsc.PackFormat.INTERLEAVED)
                pp = (k0 * qx[row, pl.ds(dd * LANES, LANES)]
                      + k1 * qx[row, pl.ds(D // 2 + dd * LANES, LANES)])
                acc = pp if acc is None else acc + pp
            abuf[e, :] = acc
        cols = [plsc.load_gather(abuf, [lanes, jnp.full((LANES,), l, jnp.int32)])
                for l in range(LANES)]
        while len(cols) > 1:
            cols = [a + b for a, b in zip(cols[::2], cols[1::2])]
        asig = 1.0 / (1.0 + jnp.exp(cols[0] * (-SCALE)))
        for e in range(LANES):
            av = asig[e]
            row = base + e
            for dd in range(D // (2 * LANES)):
                vb = plsc.bitcast(
                    kvx[row, pl.ds(D // 2 + dd * LANES, LANES)], jnp.bfloat16)
                v0, v1 = plsc.unpack(vb, format=plsc.PackFormat.INTERLEAVED)
                mx[row, pl.ds(dd * LANES, LANES)] = av * v0
                mx[row, pl.ds(D // 2 + dd * LANES, LANES)] = av * v1

    def _half(t, sqc, kvc, qc, semic, semkc, semqc,
              sqn, kvn, qn, semin, semkn, semqn,
              m_cur, dst_cur, semd_cur, sems_cur):
        @pl.when(t + 2 < NFULL)
        def _start_next():
            _wait_adjust_idx(sqn, semin, CHUNK)
            _issue_g(sqn, kvn, qn, semkn, semqn, CHUNK)

        _wait_g(sqc, kvc, qc, semkc, semqc, CHUNK)

        @pl.when(t + 4 < NFULL)
        def _prefetch_idx():
            _issue_idx(t + 4, sqc, semic, CHUNK)

        @pl.when(t >= 2)
        def _drain_scatter():
            pltpu.make_async_copy(m_cur, acc_sh.at[dst_cur], sems_cur).wait()

        _issue_dst(t, dst_cur, semd_cur, CHUNK)
        _group(kvc, qc, m_cur, 0)
        _group(kvc, qc, m_cur, LANES)
        pltpu.make_async_copy(ui_hbm.at[pl.ds(0, CHUNK)], dst_cur,
                              semd_cur).wait()
        pltpu.async_copy(m_cur, acc_sh.at[dst_cur], sems_cur, add=True)

    _issue_idx(0, sq_a, semi_a, CHUNK)
    _issue_idx(1, sq_b, semi_b, CHUNK)
    _issue_idx(2, sq_c, semi_c, CHUNK)
    _issue_idx(3, sq_d, semi_d, CHUNK)
    _wait_adjust_idx(sq_a, semi_a, CHUNK)
    _issue_g(sq_a, kv_a, q_a, semk_a, semq_a, CHUNK)
    _wait_adjust_idx(sq_b, semi_b, CHUNK)
    _issue_g(sq_b, kv_b, q_b, semk_b, semq_b, CHUNK)

    sets = ((sq_a, kv_a, q_a, semi_a, semk_a, semq_a),
            (sq_b, kv_b, q_b, semi_b, semk_b, semq_b),
            (sq_c, kv_c, q_c, semi_c, semk_c, semq_c),
            (sq_d, kv_d, q_d, semi_d, semk_d, semq_d))
    mset = ((m_a, dst_a, semd_a, sems_a), (m_b, dst_b, semd_b, sems_b))

    def _quad(j, carry):
        t0 = 4 * j
        for b in range(4):
            _half(t0 + b, *sets[b], *sets[(b + 2) % 4], *mset[b % 2])
        return carry

    lax.fori_loop(0, NFULL // 4, _quad, 0)
    pltpu.make_async_copy(m_a, acc_sh.at[dst_a], sems_a).wait()
    pltpu.make_async_copy(m_b, acc_sh.at[dst_b], sems_b).wait()

    if PARTIAL:
        _issue_idx(NFULL, sq_p, semi_a, PARTIAL)
        _issue_dst(NFULL, dst_p, semd_a, PARTIAL)
        _wait_adjust_idx(sq_p, semi_a, PARTIAL)
        _issue_g(sq_p, kv_a, q_a, semk_a, semq_a, PARTIAL)
        _wait_g(sq_p, kv_a, q_a, semk_a, semq_a, PARTIAL)
        pltpu.make_async_copy(ui_hbm.at[pl.ds(0, PARTIAL)], dst_p,
                              semd_a).wait()
        _group(kv_a, q_a, m_a, 0)
        pltpu.sync_copy(m_a.at[pl.ds(0, PARTIAL)], acc_sh.at[dst_p], add=True)

    plsc.subcore_barrier()

    pltpu.sync_copy(acc_sh.at[pl.ds(r0, RPT)],
                    out_hbm.at[pl.ds(c * N + r0, RPT)])

    @pl.when(s == 0)
    def _write_tail():
        pltpu.sync_copy(acc_sh.at[pl.ds(NSUB * RPT, TAIL)],
                        out_hbm.at[pl.ds(c * N + NSUB * RPT, TAIL)])


_sc_call = functools.partial(
    pl.kernel,
    out_type=jax.ShapeDtypeStruct((2 * N, D), jnp.float32),
    mesh=plsc.VectorSubcoreMesh(core_axis_name="c", subcore_axis_name="s",
                                num_cores=NCORE, num_subcores=NSUB),
    compiler_params=pltpu.CompilerParams(needs_layout_passes=False),
    scratch_types=[
        pltpu.VMEM((2, CHUNK), jnp.int32),
        pltpu.VMEM((2, CHUNK), jnp.int32),
        pltpu.VMEM((2, CHUNK), jnp.int32),
        pltpu.VMEM((2, CHUNK), jnp.int32),
        pltpu.VMEM((CHUNK,), jnp.int32),
        pltpu.VMEM((CHUNK,), jnp.int32),
        pltpu.VMEM((CHUNK, D), jnp.float32),
        pltpu.VMEM((CHUNK, D), jnp.float32),
        pltpu.VMEM((CHUNK, D), jnp.float32),
        pltpu.VMEM((CHUNK, D), jnp.float32),
        pltpu.VMEM((CHUNK, D), jnp.float32),
        pltpu.VMEM((CHUNK, D), jnp.float32),
        pltpu.VMEM((CHUNK, D), jnp.float32),
        pltpu.VMEM((CHUNK, D), jnp.float32),
        pltpu.VMEM((CHUNK, D), jnp.float32),
        pltpu.VMEM((CHUNK, D), jnp.float32),
        pltpu.VMEM((2, LANES), jnp.int32),
        pltpu.VMEM((LANES,), jnp.int32),
        pltpu.VMEM((LANES, LANES), jnp.float32),
        pltpu.VMEM_SHARED((N, D), jnp.float32),
        pltpu.SemaphoreType.DMA,
        pltpu.SemaphoreType.DMA,
        pltpu.SemaphoreType.DMA,
        pltpu.SemaphoreType.DMA,
        pltpu.SemaphoreType.DMA,
        pltpu.SemaphoreType.DMA,
        pltpu.SemaphoreType.DMA,
        pltpu.SemaphoreType.DMA,
        pltpu.SemaphoreType.DMA,
        pltpu.SemaphoreType.DMA,
        pltpu.SemaphoreType.DMA,
        pltpu.SemaphoreType.DMA,
        pltpu.SemaphoreType.DMA,
        pltpu.SemaphoreType.DMA,
        pltpu.SemaphoreType.DMA,
        pltpu.SemaphoreType.DMA,
    ],
)(_sc_body)


def kernel(x_user, x_item, edge_index_ui, edge_index_iu,
           W_K_ui, b_K_ui, W_Q_ui, b_Q_ui, W_V_ui, b_V_ui,
           W_K_iu, b_K_iu, W_Q_iu, b_Q_iu, W_V_iu, b_V_iu,
           W_O, b_O, ln_gamma, ln_beta):
    f32 = jnp.float32
    wk = jnp.stack([W_K_ui, W_K_iu])
    wq = jnp.stack([W_Q_ui, W_Q_iu])
    wv = jnp.stack([W_V_ui, W_V_iu])
    bk = jnp.stack([b_K_ui, b_K_iu]).reshape(2, 1, D)
    bq = jnp.stack([b_Q_ui, b_Q_iu]).reshape(2, 1, D)
    bv = jnp.stack([b_V_ui, b_V_iu]).reshape(2, 1, D)

    wspec = pl.BlockSpec((1, D, D), lambda i: (i // HALF_BLKS, 0, 0))
    bspec = pl.BlockSpec((1, 1, D), lambda i: (i // HALF_BLKS, 0, 0))
    rspec = pl.BlockSpec((BLK, D), lambda i: (i, 0))
    hspec = pl.BlockSpec((BLK, D), lambda i: (i % HALF_BLKS, 0))
    kv, q = pl.pallas_call(
        _proj_body,
        grid=(NBLK,),
        in_specs=[hspec, hspec, wspec, bspec, wspec, bspec, wspec, bspec],
        out_specs=[rspec, rspec],
        out_shape=[jax.ShapeDtypeStruct((2 * N, D), f32),
                   jax.ShapeDtypeStruct((2 * N, D), f32)],
    )(x_user.astype(f32), x_item.astype(f32), wk, bk, wq, bq, wv, bv)

    i32 = jnp.int32
    acc = _sc_call(kv, q,
                   edge_index_ui.astype(i32).reshape(-1),
                   edge_index_iu.astype(i32).reshape(-1))

    vspec = pl.BlockSpec((1, D), lambda i: (0, 0))
    h = pl.pallas_call(
        _out_body,
        grid=(NBLK,),
        in_specs=[rspec, pl.BlockSpec((D, D), lambda i: (0, 0)),
                  vspec, vspec, vspec],
        out_specs=rspec,
        out_shape=jax.ShapeDtypeStruct((2 * N, D), f32),
    )(acc, W_O.astype(f32), b_O.reshape(1, D), ln_gamma.reshape(1, D),
      ln_beta.reshape(1, D))

    return h[N:], h[:N]
